# Initial kernel scaffold; baseline (speedup 1.0000x reference)
import jax
import jax.numpy as jnp
from jax.experimental import pallas as pl


def kernel(x, y):
    raise NotImplementedError("write your pallas kernel here")


# speedup vs baseline: 114.0434x; 114.0434x over previous
import functools

import jax
import jax.numpy as jnp
from jax import lax
from jax.experimental import pallas as pl
from jax.experimental.pallas import tpu as pltpu
from jax.experimental.pallas import tpu_sc as plsc

N = 16777216
NUM_WORKERS = 32
PER_WORKER = N // NUM_WORKERS
CHUNK = 16384
NCHUNKS = PER_WORKER // CHUNK
UNROLL = 8
VREGS_PER_CHUNK = CHUNK // 16
BINS = 256
MIN_VAL = -4.0
INV_WIDTH = 32.0


def _hist_body(x_hbm, y_hbm, out_hbm, buf, hist, local, sem):
    wid = lax.axis_index("s") * 2 + lax.axis_index("c")
    base = wid * PER_WORKER

    zeros16 = jnp.zeros((16,), jnp.float32)
    ones16 = jnp.ones((16,), jnp.float32)
    lane = lax.broadcasted_iota(jnp.int32, (16,), 0)

    def zbody(i, _):
        hist[pl.ds(i * 16, 16)] = zeros16
        return 0

    lax.fori_loop(0, 512, zbody, 0)

    for which, src in enumerate((x_hbm, y_hbm)):
        laneoff = lane * 256 + which * 4096

        def cbody(c, _, src=src, laneoff=laneoff):
            pltpu.sync_copy(src.at[pl.ds(base + c * CHUNK, CHUNK)], buf)

            def vbody(i, _):
                for k in range(UNROLL):
                    v = buf[pl.ds((i * UNROLL + k) * 16, 16)]
                    t = (v - MIN_VAL) * INV_WIDTH
                    idx = t.astype(jnp.int32)
                    idx = jnp.minimum(jnp.maximum(idx, 0), BINS - 1)
                    mask = (v >= -4.0) & (v <= 4.0)
                    plsc.addupdate_scatter(hist, [idx + laneoff], ones16,
                                           mask=mask)
                return 0

            lax.fori_loop(0, VREGS_PER_CHUNK // UNROLL, vbody, 0)
            return 0

        lax.fori_loop(0, NCHUNKS, cbody, 0)

    for which in range(2):
        for j in range(BINS // 16):
            acc = zeros16
            for l in range(16):
                acc = acc + hist[pl.ds(which * 4096 + l * 256 + j * 16, 16)]
            local[pl.ds(which * 256 + j * 16, 16)] = acc

    pltpu.sync_copy(local, out_hbm.at[wid])


def _make_hist_kernel():
    mesh = plsc.VectorSubcoreMesh(core_axis_name="c", subcore_axis_name="s")
    return pl.kernel(
        _hist_body,
        mesh=mesh,
        compiler_params=pltpu.CompilerParams(needs_layout_passes=False),
        out_type=jax.ShapeDtypeStruct((NUM_WORKERS, 512), jnp.float32),
        scratch_types=[
            pltpu.VMEM((CHUNK,), jnp.float32),
            pltpu.VMEM((8192,), jnp.float32),
            pltpu.VMEM((512,), jnp.float32),
            pltpu.SemaphoreType.DMA,
        ],
    )


def _mi_body(p_ref, o_ref):
    p = p_ref[...]
    s = jnp.sum(p, axis=0, keepdims=True)
    hx = s[:, :BINS]
    hy = s[:, BINS:]
    sx = jnp.sum(hx)
    sy = jnp.sum(hy)
    jp = (hx + hy) / (sx + sy)
    px = hx / sx
    py = hy / sy
    ljp = jnp.log(jp)
    lpx = jnp.log(px)
    lpy = jnp.log(py)
    a = jp * (ljp - lpy)
    mi = float(BINS) * jnp.sum(a) - jnp.sum(lpx) * jnp.sum(jp)
    o_ref[...] = jnp.reshape(-mi, (1, 1))


def _mi_reduce(partials):
    return pl.pallas_call(
        _mi_body,
        out_shape=jax.ShapeDtypeStruct((1, 1), jnp.float32),
    )(partials)


@jax.jit
def kernel(x, y):
    partials = _make_hist_kernel()(x, y)
    out = _mi_reduce(partials)
    return out[0, 0]


# speedup vs baseline: 132.6462x; 1.1631x over previous
import jax
import jax.numpy as jnp
from jax import lax
from jax.experimental import pallas as pl
from jax.experimental.pallas import tpu as pltpu
from jax.experimental.pallas import tpu_sc as plsc

N = 16777216
NUM_WORKERS = 32
PER_WORKER = N // NUM_WORKERS
CHUNK = 32768
NCHUNKS = PER_WORKER // CHUNK
UNROLL = 8
VREGS_PER_CHUNK = CHUNK // 16
BINS = 256


def _hist_body(x_hbm, y_hbm, out_hbm, buf0, buf1, hist, local, sem0, sem1):
    wid = lax.axis_index("s") * 2 + lax.axis_index("c")
    base = wid * PER_WORKER
    sems = (sem0, sem1)
    bufs_all = (buf0, buf1)

    zeros16 = jnp.zeros((16,), jnp.float32)
    ones16 = jnp.ones((16,), jnp.float32)
    lane = lax.broadcasted_iota(jnp.int32, (16,), 0)

    def zbody(i, _):
        hist[pl.ds(i * 16, 16)] = zeros16
        return 0

    lax.fori_loop(0, 512, zbody, 0)

    for which, src in enumerate((x_hbm, y_hbm)):
        lanewhich = lane + which * 4096

        pltpu.async_copy(src.at[pl.ds(base, CHUNK)], buf0, sem0)

        def obody(g, _, src=src, lanewhich=lanewhich):
            for s in range(2):
                c = g * 2 + s
                pltpu.make_async_copy(
                    src.at[pl.ds(base, CHUNK)], bufs_all[s], sems[s]
                ).wait()

                @pl.when(c + 1 < NCHUNKS)
                def _start(s=s, c=c, src=src):
                    pltpu.async_copy(
                        src.at[pl.ds(base + (c + 1) * CHUNK, CHUNK)],
                        bufs_all[s ^ 1],
                        sems[s ^ 1],
                    )

                bufs = bufs_all[s]

                def vbody(i, _, bufs=bufs, lanewhich=lanewhich):
                    for k in range(UNROLL):
                        v = bufs[pl.ds((i * UNROLL + k) * 16, 16)]
                        t = (v + 4.0) * 32.0
                        u = t.astype(jnp.int32)
                        mask = plsc.bitcast(t, jnp.uint32) <= jnp.uint32(0x43800000)
                        idx = jnp.minimum(u, BINS - 1)
                        flat = (idx << 4) + lanewhich
                        plsc.addupdate_scatter(hist, [flat], ones16, mask=mask)
                    return 0

                lax.fori_loop(0, VREGS_PER_CHUNK // UNROLL, vbody, 0)
            return 0

        lax.fori_loop(0, NCHUNKS // 2, obody, 0)

    for which in range(2):
        base_vec = lane * 16 + which * 4096
        for j in range(BINS // 16):
            acc = zeros16
            for l in range(16):
                acc = acc + plsc.load_gather(hist, [base_vec + (j * 256 + l)])
            local[pl.ds(which * 256 + j * 16, 16)] = acc

    pltpu.sync_copy(local, out_hbm.at[wid])


def _make_hist_kernel():
    mesh = plsc.VectorSubcoreMesh(core_axis_name="c", subcore_axis_name="s")
    return pl.kernel(
        _hist_body,
        mesh=mesh,
        compiler_params=pltpu.CompilerParams(needs_layout_passes=False),
        out_type=jax.ShapeDtypeStruct((NUM_WORKERS, 512), jnp.float32),
        scratch_types=[
            pltpu.VMEM((CHUNK,), jnp.float32),
            pltpu.VMEM((CHUNK,), jnp.float32),
            pltpu.VMEM((8192,), jnp.float32),
            pltpu.VMEM((512,), jnp.float32),
            pltpu.SemaphoreType.DMA,
            pltpu.SemaphoreType.DMA,
        ],
    )


def _mi_body(p_ref, o_ref):
    p = p_ref[...]
    s = jnp.sum(p, axis=0, keepdims=True)
    hx = s[:, :BINS]
    hy = s[:, BINS:]
    sx = jnp.sum(hx)
    sy = jnp.sum(hy)
    jp = (hx + hy) / (sx + sy)
    px = hx / sx
    py = hy / sy
    ljp = jnp.log(jp)
    lpx = jnp.log(px)
    lpy = jnp.log(py)
    a = jp * (ljp - lpy)
    mi = float(BINS) * jnp.sum(a) - jnp.sum(lpx) * jnp.sum(jp)
    o_ref[...] = jnp.reshape(-mi, (1, 1))


def _mi_reduce(partials):
    return pl.pallas_call(
        _mi_body,
        out_shape=jax.ShapeDtypeStruct((1, 1), jnp.float32),
    )(partials)


@jax.jit
def kernel(x, y):
    partials = _make_hist_kernel()(x, y)
    out = _mi_reduce(partials)
    return out[0, 0]
